# baseline (speedup 1.0000x reference)
import numpy as np
import jax
import jax.numpy as jnp
from jax.experimental import pallas as pl
from jax.experimental.pallas import tpu as pltpu

_ALPHA = 4
_G = 8


def _slow_idx(t: int) -> list:
    n = t // _ALPHA
    return [int(v) for v in np.linspace(0.0, t - 1, n).astype(np.int32)]


def kernel(frames):
    c, t, h, w = frames.shape
    idx = _slow_idx(t)
    n = len(idx)
    g_sz = _G if t % _G == 0 else _ALPHA
    gpc = t // g_sz
    spg = g_sz // _ALPHA
    k_steps = c * gpc
    offs = [
        [idx[g * spg + s] - g_sz * g for s in range(spg)] for g in range(gpc)
    ]
    assert all(0 <= o < g_sz for row in offs for o in row)

    def body(in_hbm, slow_hbm, fast_hbm, buf, sem_in, sem_fast, sem_slow):
        j = pl.program_id(0)
        par = jax.lax.rem(j, 2)
        nxt = jax.lax.rem(j + 1, 2)

        def split(step):
            return jax.lax.div(step, gpc), jax.lax.rem(step, gpc)

        def in_copy(step, slot):
            ch, g = split(step)
            return pltpu.make_async_copy(
                in_hbm.at[ch, pl.ds(g * g_sz, g_sz)],
                buf.at[slot],
                sem_in.at[slot],
            )

        def fast_copy(step, slot):
            ch, g = split(step)
            return pltpu.make_async_copy(
                buf.at[slot],
                fast_hbm.at[ch, pl.ds(g * g_sz, g_sz)],
                sem_fast.at[slot],
            )

        def slow_copy(step, slot, s):
            ch, g = split(step)
            off = jnp.int32(offs[0][s])
            for k in range(1, gpc):
                off = jnp.where(g == k, jnp.int32(offs[k][s]), off)
            return pltpu.make_async_copy(
                buf.at[slot, off],
                slow_hbm.at[ch, g * spg + s],
                sem_slow.at[slot, s],
            )

        @pl.when(j == 0)
        def _():
            in_copy(j, par).start()

        @pl.when(j > 0)
        def _():
            fast_copy(j - 1, nxt).wait()
            for s in range(spg):
                slow_copy(j - 1, nxt, s).wait()

        @pl.when(j + 1 < k_steps)
        def _():
            in_copy(j + 1, nxt).start()

        in_copy(j, par).wait()
        fast_copy(j, par).start()
        for s in range(spg):
            slow_copy(j, par, s).start()

        @pl.when(j == k_steps - 1)
        def _():
            fast_copy(j, par).wait()
            for s in range(spg):
                slow_copy(j, par, s).wait()

    slow, fast = pl.pallas_call(
        body,
        grid=(k_steps,),
        in_specs=[pl.BlockSpec(memory_space=pl.ANY)],
        out_specs=[
            pl.BlockSpec(memory_space=pl.ANY),
            pl.BlockSpec(memory_space=pl.ANY),
        ],
        out_shape=[
            jax.ShapeDtypeStruct((c, n, h, w), frames.dtype),
            jax.ShapeDtypeStruct((c, t, h, w), frames.dtype),
        ],
        scratch_shapes=[
            pltpu.VMEM((2, g_sz, h, w), frames.dtype),
            pltpu.SemaphoreType.DMA((2,)),
            pltpu.SemaphoreType.DMA((2,)),
            pltpu.SemaphoreType.DMA((2, spg)),
        ],
    )(frames)
    return (slow, fast)


# speedup vs baseline: 1.2479x; 1.2479x over previous
import numpy as np
import jax
import jax.numpy as jnp
from jax.experimental import pallas as pl
from jax.experimental.pallas import tpu as pltpu

_ALPHA = 4
_G = 16


def _slow_idx(t: int) -> list:
    n = t // _ALPHA
    return [int(v) for v in np.linspace(0.0, t - 1, n).astype(np.int32)]


def kernel(frames):
    c, t, h, w = frames.shape
    idx = _slow_idx(t)
    n = len(idx)
    g_sz = _G if t % _G == 0 else _ALPHA
    gpc = t // g_sz
    spg = g_sz // _ALPHA
    k_steps = c * gpc
    offs = [
        [idx[g * spg + s] - g_sz * g for s in range(spg)] for g in range(gpc)
    ]
    assert all(0 <= o < g_sz for row in offs for o in row)

    def body(in_hbm, slow_hbm, fast_hbm, buf, sem_in, sem_fast, sem_slow):
        j = pl.program_id(0)
        par = jax.lax.rem(j, 2)
        nxt = jax.lax.rem(j + 1, 2)

        def split(step):
            return jax.lax.div(step, gpc), jax.lax.rem(step, gpc)

        def in_copy(step, slot):
            ch, g = split(step)
            return pltpu.make_async_copy(
                in_hbm.at[ch, pl.ds(g * g_sz, g_sz)],
                buf.at[slot],
                sem_in.at[slot],
            )

        def fast_copy(step, slot):
            ch, g = split(step)
            return pltpu.make_async_copy(
                buf.at[slot],
                fast_hbm.at[ch, pl.ds(g * g_sz, g_sz)],
                sem_fast.at[slot],
            )

        def slow_copy(step, slot, s):
            ch, g = split(step)
            off = jnp.int32(offs[0][s])
            for k in range(1, gpc):
                off = jnp.where(g == k, jnp.int32(offs[k][s]), off)
            return pltpu.make_async_copy(
                buf.at[slot, off],
                slow_hbm.at[ch, g * spg + s],
                sem_slow.at[slot, s],
            )

        @pl.when(j == 0)
        def _():
            in_copy(j, par).start()

        @pl.when(j > 0)
        def _():
            fast_copy(j - 1, nxt).wait()
            for s in range(spg):
                slow_copy(j - 1, nxt, s).wait()

        @pl.when(j + 1 < k_steps)
        def _():
            in_copy(j + 1, nxt).start()

        in_copy(j, par).wait()
        fast_copy(j, par).start()
        for s in range(spg):
            slow_copy(j, par, s).start()

        @pl.when(j == k_steps - 1)
        def _():
            fast_copy(j, par).wait()
            for s in range(spg):
                slow_copy(j, par, s).wait()

    slow, fast = pl.pallas_call(
        body,
        grid=(k_steps,),
        in_specs=[pl.BlockSpec(memory_space=pl.ANY)],
        out_specs=[
            pl.BlockSpec(memory_space=pl.ANY),
            pl.BlockSpec(memory_space=pl.ANY),
        ],
        out_shape=[
            jax.ShapeDtypeStruct((c, n, h, w), frames.dtype),
            jax.ShapeDtypeStruct((c, t, h, w), frames.dtype),
        ],
        scratch_shapes=[
            pltpu.VMEM((2, g_sz, h, w), frames.dtype),
            pltpu.SemaphoreType.DMA((2,)),
            pltpu.SemaphoreType.DMA((2,)),
            pltpu.SemaphoreType.DMA((2, spg)),
        ],
    )(frames)
    return (slow, fast)


# speedup vs baseline: 1.3456x; 1.0782x over previous
import numpy as np
import jax
import jax.numpy as jnp
from jax.experimental import pallas as pl
from jax.experimental.pallas import tpu as pltpu

_ALPHA = 4
_G = 32


def _slow_idx(t: int) -> list:
    n = t // _ALPHA
    return [int(v) for v in np.linspace(0.0, t - 1, n).astype(np.int32)]


def kernel(frames):
    c, t, h, w = frames.shape
    idx = _slow_idx(t)
    n = len(idx)
    g_sz = _G if t % _G == 0 else _ALPHA
    gpc = t // g_sz
    spg = g_sz // _ALPHA
    k_steps = c * gpc
    offs = [
        [idx[g * spg + s] - g_sz * g for s in range(spg)] for g in range(gpc)
    ]
    assert all(0 <= o < g_sz for row in offs for o in row)

    def body(in_hbm, slow_hbm, fast_hbm, buf, sem_in, sem_fast, sem_slow):
        j = pl.program_id(0)
        par = jax.lax.rem(j, 2)
        nxt = jax.lax.rem(j + 1, 2)

        def split(step):
            return jax.lax.div(step, gpc), jax.lax.rem(step, gpc)

        def in_copy(step, slot):
            ch, g = split(step)
            return pltpu.make_async_copy(
                in_hbm.at[ch, pl.ds(g * g_sz, g_sz)],
                buf.at[slot],
                sem_in.at[slot],
            )

        def fast_copy(step, slot):
            ch, g = split(step)
            return pltpu.make_async_copy(
                buf.at[slot],
                fast_hbm.at[ch, pl.ds(g * g_sz, g_sz)],
                sem_fast.at[slot],
            )

        def slow_copy(step, slot, s):
            ch, g = split(step)
            off = jnp.int32(offs[0][s])
            for k in range(1, gpc):
                off = jnp.where(g == k, jnp.int32(offs[k][s]), off)
            return pltpu.make_async_copy(
                buf.at[slot, off],
                slow_hbm.at[ch, g * spg + s],
                sem_slow.at[slot, s],
            )

        @pl.when(j == 0)
        def _():
            in_copy(j, par).start()

        @pl.when(j > 0)
        def _():
            fast_copy(j - 1, nxt).wait()
            for s in range(spg):
                slow_copy(j - 1, nxt, s).wait()

        @pl.when(j + 1 < k_steps)
        def _():
            in_copy(j + 1, nxt).start()

        in_copy(j, par).wait()
        fast_copy(j, par).start()
        for s in range(spg):
            slow_copy(j, par, s).start()

        @pl.when(j == k_steps - 1)
        def _():
            fast_copy(j, par).wait()
            for s in range(spg):
                slow_copy(j, par, s).wait()

    slow, fast = pl.pallas_call(
        body,
        grid=(k_steps,),
        in_specs=[pl.BlockSpec(memory_space=pl.ANY)],
        out_specs=[
            pl.BlockSpec(memory_space=pl.ANY),
            pl.BlockSpec(memory_space=pl.ANY),
        ],
        out_shape=[
            jax.ShapeDtypeStruct((c, n, h, w), frames.dtype),
            jax.ShapeDtypeStruct((c, t, h, w), frames.dtype),
        ],
        scratch_shapes=[
            pltpu.VMEM((2, g_sz, h, w), frames.dtype),
            pltpu.SemaphoreType.DMA((2,)),
            pltpu.SemaphoreType.DMA((2,)),
            pltpu.SemaphoreType.DMA((2, spg)),
        ],
    )(frames)
    return (slow, fast)


# speedup vs baseline: 1.5630x; 1.1616x over previous
import numpy as np
import jax
import jax.numpy as jnp
from jax.experimental import pallas as pl

_ALPHA = 4
_G = 32


def _slow_idx(t: int) -> list:
    n = t // _ALPHA
    return [int(v) for v in np.linspace(0.0, t - 1, n).astype(np.int32)]


def kernel(frames):
    c, t, h, w = frames.shape
    idx = _slow_idx(t)
    n = len(idx)
    g_sz = _G if t % _G == 0 else _ALPHA
    gpc = t // g_sz
    spg = g_sz // _ALPHA
    offs = [
        [idx[g * spg + s] - g_sz * g for s in range(spg)] for g in range(gpc)
    ]
    assert all(0 <= o < g_sz for row in offs for o in row)

    def body(in_ref, slow_ref, fast_ref):
        fast_ref[...] = in_ref[...]
        if gpc == 1:
            for s in range(spg):
                slow_ref[:, s : s + 1] = in_ref[:, offs[0][s] : offs[0][s] + 1]
        else:
            j = pl.program_id(0)
            g = jax.lax.rem(j, gpc)
            for s in range(spg):
                off = jnp.int32(offs[0][s])
                for k in range(1, gpc):
                    off = jnp.where(g == k, jnp.int32(offs[k][s]), off)
                slow_ref[:, s : s + 1] = in_ref[:, pl.ds(off, 1)]

    slow, fast = pl.pallas_call(
        body,
        grid=(c * gpc,),
        in_specs=[
            pl.BlockSpec((1, g_sz, h, w), lambda j: (j // gpc, j % gpc, 0, 0))
        ],
        out_specs=[
            pl.BlockSpec((1, spg, h, w), lambda j: (j // gpc, j % gpc, 0, 0)),
            pl.BlockSpec((1, g_sz, h, w), lambda j: (j // gpc, j % gpc, 0, 0)),
        ],
        out_shape=[
            jax.ShapeDtypeStruct((c, n, h, w), frames.dtype),
            jax.ShapeDtypeStruct((c, t, h, w), frames.dtype),
        ],
    )(frames)
    return (slow, fast)
